# Initial kernel scaffold; baseline (speedup 1.0000x reference)
import jax
import jax.numpy as jnp
from jax.experimental import pallas as pl


def kernel(outputs, targets):
    raise NotImplementedError("write your pallas kernel here")


# speedup vs baseline: 53.9260x; 53.9260x over previous
import functools

import jax
import jax.numpy as jnp
import numpy as np
from jax.experimental import pallas as pl

_ANCHORS = np.array(
    [[1.3221, 1.73145], [3.19275, 4.00944], [5.05587, 8.09892],
     [9.47112, 4.84053], [11.2364, 10.0071]], dtype=np.float32)
_A = 5
_NC = 20
_NEG = -1e30
_INTERPRET = False


def _sigmoid(x):
    return 1.0 / (1.0 + jnp.exp(-x))


def _prior_best_anchor(gx, gy, gw, gh):
    pcx = jnp.floor(gx) + 0.5
    pcy = jnp.floor(gy) + 0.5
    bx1 = gx - gw * 0.5
    bx2 = gx + gw * 0.5
    by1 = gy - gh * 0.5
    by2 = gy + gh * 0.5
    area_b = (bx2 - bx1) * (by2 - by1)
    best = jnp.full(gx.shape, -1.0, jnp.float32)
    ai = jnp.zeros(gx.shape, jnp.int32)
    for a in range(_A):
        wa = float(_ANCHORS[a, 0])
        ha = float(_ANCHORS[a, 1])
        ax1 = pcx - wa * 0.5
        ax2 = pcx + wa * 0.5
        ay1 = pcy - ha * 0.5
        ay2 = pcy + ha * 0.5
        area_a = (ax2 - ax1) * (ay2 - ay1)
        iw = jnp.maximum(jnp.minimum(ax2, bx2) - jnp.maximum(ax1, bx1), 0.0)
        ih = jnp.maximum(jnp.minimum(ay2, by2) - jnp.maximum(ay1, by1), 0.0)
        inter = iw * ih
        union = area_a + area_b - inter
        iou = inter / jnp.maximum(union, 1e-12)
        upd = iou > best
        best = jnp.where(upd, iou, best)
        ai = jnp.where(upd, jnp.int32(a), ai)
    return ai


def _yolo_kernel(o_ref, tT_ref, tC_ref, out_ref, *, H, W, N, B):
    b = pl.program_id(0)
    HW = H * W
    f32 = jnp.float32

    cls_r = tT_ref[0, 0:1, :]
    gx_r = tT_ref[0, 1:2, :] * W
    gy_r = tT_ref[0, 2:3, :] * H
    gw_r = tT_ref[0, 3:4, :] * W
    gh_r = tT_ref[0, 4:5, :] * H
    sraw_r = (tT_ref[0, 0:1, :] + tT_ref[0, 1:2, :] + tT_ref[0, 2:3, :]
              + tT_ref[0, 3:4, :] + tT_ref[0, 4:5, :])

    gx_c = tC_ref[0, :, 1:2] * W
    gy_c = tC_ref[0, :, 2:3] * H
    gw_c = tC_ref[0, :, 3:4] * W
    gh_c = tC_ref[0, :, 4:5] * H

    gt_num = jnp.sum((sraw_r > 0.0).astype(f32), axis=1, keepdims=True)
    lane_r = jax.lax.broadcasted_iota(jnp.int32, (1, N), 1).astype(f32)
    lane_c = jax.lax.broadcasted_iota(jnp.int32, (N, 1), 0).astype(f32)
    valid_r = lane_r < gt_num
    valid_c = lane_c < gt_num

    bx1_c = gx_c - gw_c * 0.5
    bx2_c = gx_c + gw_c * 0.5
    by1_c = gy_c - gh_c * 0.5
    by2_c = gy_c + gh_c * 0.5
    area_b_c = (bx2_c - bx1_c) * (by2_c - by1_c)

    cxf_r = jnp.floor(gx_r)
    cyf_r = jnp.floor(gy_r)
    cell_r = (cyf_r * W + cxf_r).astype(jnp.int32)
    cell_c = (jnp.floor(gy_c) * W + jnp.floor(gx_c)).astype(jnp.int32)
    ai_r = _prior_best_anchor(gx_r, gy_r, gw_r, gh_r)
    ai_c = _prior_best_anchor(gx_c, gy_c, gw_c, gh_c)

    key_r = cell_r * 8 + ai_r
    key_c = cell_c * 8 + ai_c
    ii = jax.lax.broadcasted_iota(jnp.int32, (N, N), 1)
    jj = jax.lax.broadcasted_iota(jnp.int32, (N, N), 0)
    dup_later = ((key_c == key_r) & (jj > ii) & valid_c).astype(f32)
    taken = jnp.max(dup_later, axis=0, keepdims=True)
    winner = valid_r.astype(f32) * (1.0 - taken)

    lane_f = jax.lax.broadcasted_iota(jnp.int32, (1, HW), 1).astype(f32)
    ys = jnp.floor(lane_f / W)
    xs = lane_f - ys * W

    s_all = jnp.zeros((1, 1), f32)
    s_ign = jnp.zeros((1, 1), f32)
    gmax = jnp.full((1, 1), _NEG, f32)
    for a in range(_A):
        base = a * (5 + _NC)
        tx = o_ref[0, base + 0:base + 1, :]
        ty = o_ref[0, base + 1:base + 2, :]
        tw = o_ref[0, base + 2:base + 3, :]
        th = o_ref[0, base + 3:base + 4, :]
        tc = o_ref[0, base + 4:base + 5, :]
        px = _sigmoid(tx) + xs
        py = _sigmoid(ty) + ys
        pw = jnp.exp(tw) * float(_ANCHORS[a, 0])
        ph = jnp.exp(th) * float(_ANCHORS[a, 1])
        ax1 = px - pw * 0.5
        ax2 = px + pw * 0.5
        ay1 = py - ph * 0.5
        ay2 = py + ph * 0.5
        area_a = (ax2 - ax1) * (ay2 - ay1)
        conf = _sigmoid(tc)
        c2 = conf * conf
        s_all = s_all + jnp.sum(c2, axis=1, keepdims=True)
        iw = jnp.maximum(jnp.minimum(ax2, bx2_c) - jnp.maximum(ax1, bx1_c), 0.0)
        ih = jnp.maximum(jnp.minimum(ay2, by2_c) - jnp.maximum(ay1, by1_c), 0.0)
        inter = iw * ih
        d = inter + inter - (area_a + area_b_c - inter)
        dm = jnp.where(valid_c, d, _NEG)
        dmax = jnp.max(dm, axis=0, keepdims=True)
        ign = (dmax >= 0.0).astype(f32)
        s_ign = s_ign + jnp.sum(ign * c2, axis=1, keepdims=True)
        gmax = jnp.maximum(gmax, jnp.max(dmax, axis=1, keepdims=True))

    has_pos = (gmax > 0.0).astype(f32)
    gt_pos = (gt_num > 0.0).astype(f32)
    dense_iou = gt_pos * (s_all - has_pos * s_ign)

    sub676 = jax.lax.broadcasted_iota(jnp.int32, (HW, N), 0)
    onehot = (sub676 == cell_r).astype(f32)
    G = jax.lax.dot_general(o_ref[0], onehot,
                            (((1,), (0,)), ((), ())),
                            preferred_element_type=f32)

    m = [(ai_r == a).astype(f32) for a in range(_A)]

    def sel(c):
        acc = m[0] * G[c:c + 1, :]
        for a in range(1, _A):
            r = a * (5 + _NC) + c
            acc = acc + m[a] * G[r:r + 1, :]
        return acc

    txs = sel(0)
    tys = sel(1)
    tws = sel(2)
    ths = sel(3)
    tcs = sel(4)
    wa_sel = sum(m[a] * float(_ANCHORS[a, 0]) for a in range(_A))
    ha_sel = sum(m[a] * float(_ANCHORS[a, 1]) for a in range(_A))

    sxs = _sigmoid(txs)
    sys_ = _sigmoid(tys)
    ew = jnp.exp(tws)
    eh = jnp.exp(ths)
    pw_s = ew * wa_sel
    ph_s = eh * ha_sel
    px_s = sxs + cxf_r
    py_s = sys_ + cyf_r

    dx = gx_r - cxf_r
    dy = gy_r - cyf_r
    dw = gw_r / wa_sel
    dh = gh_r / ha_sel
    scale = 2.0 - (pw_s / W) * (ph_s / H)
    box_term = scale * ((sxs - dx) ** 2 + (sys_ - dy) ** 2
                        + (ew - dw) ** 2 + (eh - dh) ** 2)

    pax1 = px_s - pw_s * 0.5
    pax2 = px_s + pw_s * 0.5
    pay1 = py_s - ph_s * 0.5
    pay2 = py_s + ph_s * 0.5
    area_p = (pax2 - pax1) * (pay2 - pay1)
    iw2 = jnp.maximum(jnp.minimum(pax2, bx2_c) - jnp.maximum(pax1, bx1_c), 0.0)
    ih2 = jnp.maximum(jnp.minimum(pay2, by2_c) - jnp.maximum(pay1, by1_c), 0.0)
    inter2 = iw2 * ih2
    union2 = area_p + area_b_c - inter2
    iou2 = inter2 / jnp.maximum(union2, 1e-12)
    miou = jnp.max(jnp.where(valid_c, iou2, -1.0), axis=0, keepdims=True)

    conf_s = _sigmoid(tcs)
    basew = 1.0 - has_pos * (miou >= 0.5).astype(f32)
    iou_corr = (conf_s - miou) ** 2 - basew * conf_s * conf_s

    logits = [sel(5 + k) for k in range(_NC)]
    mx = logits[0]
    for k in range(1, _NC):
        mx = jnp.maximum(mx, logits[k])
    se = jnp.exp(logits[0] - mx)
    picked = (cls_r == 0.0).astype(f32) * logits[0]
    for k in range(1, _NC):
        se = se + jnp.exp(logits[k] - mx)
        picked = picked + (cls_r == float(k)).astype(f32) * logits[k]
    ce = mx + jnp.log(se) - picked

    corr = winner * (box_term + iou_corr + 2.0 * ce)
    img_total = (dense_iou + jnp.sum(corr, axis=1, keepdims=True)) \
        * (1.0 / (2.0 * B))

    @pl.when(b == 0)
    def _():
        out_ref[...] = jnp.zeros_like(out_ref)

    out_ref[...] += jnp.broadcast_to(img_total, out_ref.shape)


def kernel(outputs, targets):
    outputs = jnp.asarray(outputs, jnp.float32)
    targets = jnp.asarray(targets, jnp.float32)
    B, C, H, W = outputs.shape
    N = targets.shape[1]
    HW = H * W
    o2 = outputs.reshape(B, C, HW)
    tC = targets
    tT = jnp.swapaxes(targets, 1, 2)

    out = pl.pallas_call(
        functools.partial(_yolo_kernel, H=H, W=W, N=N, B=B),
        grid=(B,),
        in_specs=[
            pl.BlockSpec((1, C, HW), lambda b: (b, 0, 0)),
            pl.BlockSpec((1, 5, N), lambda b: (b, 0, 0)),
            pl.BlockSpec((1, N, 5), lambda b: (b, 0, 0)),
        ],
        out_specs=pl.BlockSpec((8, 128), lambda b: (0, 0)),
        out_shape=jax.ShapeDtypeStruct((8, 128), jnp.float32),
        interpret=_INTERPRET,
    )(o2, tT, tC)
    return out[0, 0]
